# baseline (speedup 1.0000x reference)
import functools

import jax
import jax.numpy as jnp
from jax import lax
from jax.experimental import pallas as pl
from jax.experimental.pallas import tpu as pltpu
from jax.experimental.pallas import tpu_sc as plsc

N_NODES = 10000
N_PAD = 10240
N_EDGES = 320000
D_IN = 128
D_HID = 256
D_OUT = 64

NC = 2
NS = 16
CHUNK = 128
NCHUNKS = N_EDGES // CHUNK
T_STEPS = -(-NCHUNKS // (NC * NS))
ROWS_PER_TILE = N_PAD // NS
ZROWS = 128


NBUF = 2
ISLOTS = 4
W = NC * NS
T = 79
CH_PAD = W * T
E_PAD = CH_PAD * CHUNK
DUMP_ROW0 = N_NODES
NDUMP = N_PAD - N_NODES


def _make_sc_spmm(d, tiled):
    mesh = plsc.VectorSubcoreMesh(
        core_axis_name="c", subcore_axis_name="s",
        num_cores=NC, num_subcores=NS)
    scratch = {
        "srcs": [pltpu.VMEM((CHUNK,), jnp.int32) for _ in range(ISLOTS)],
        "dsts": [pltpu.VMEM((CHUNK,), jnp.int32) for _ in range(ISLOTS)],
        "rows": [pltpu.VMEM((CHUNK, d), jnp.float32) for _ in range(NBUF)],
        "acc": pltpu.VMEM_SHARED((N_PAD, d), jnp.float32),
        "isems": [pltpu.SemaphoreType.DMA for _ in range(ISLOTS)],
        "gsems": [pltpu.SemaphoreType.DMA for _ in range(NBUF)],
        "ssems": [pltpu.SemaphoreType.DMA for _ in range(NBUF)],
    }
    out_type = jax.ShapeDtypeStruct((NC, N_PAD, d), jnp.float32)

    def kern(table_hbm, src_hbm, dst_hbm, out_hbm, *, srcs, dsts, rows,
             acc, isems, gsems, ssems):
        cid = lax.axis_index("c")
        sid = lax.axis_index("s")
        wid = cid * NS + sid
        start = wid * 78 + jnp.minimum(wid, 4)
        nch = jnp.where(wid < 4, 79, 78)

        _fill(rows[0], CHUNK, d, 0.0)
        r0 = sid * ROWS_PER_TILE
        for j in range(ROWS_PER_TILE // CHUNK):
            pltpu.sync_copy(rows[0], acc.at[pl.ds(r0 + j * CHUNK, CHUNK)])
        plsc.subcore_barrier()

        def idx_load(t, slot):
            base = (start + t) * CHUNK
            pltpu.async_copy(dst_hbm.at[pl.ds(base, CHUNK)], dsts[slot],
                             isems[slot])
            pltpu.async_copy(src_hbm.at[pl.ds(base, CHUNK)], srcs[slot],
                             isems[slot])

        def idx_wait(slot):
            pltpu.make_async_copy(
                src_hbm.at[pl.ds(0, CHUNK)], dsts[slot], isems[slot]).wait()
            pltpu.make_async_copy(
                src_hbm.at[pl.ds(0, CHUNK)], srcs[slot], isems[slot]).wait()

        idx_load(0, 0)
        idx_load(1, 1)

        def step4(g, carry):
            for b in range(ISLOTS):
                t = g * ISLOTS + b
                rb = b % NBUF

                @pl.when(t < nch)
                def _():
                    idx_wait(b)

                    @pl.when(t >= NBUF)
                    def _():
                        pltpu.make_async_copy(
                            rows[rb], acc.at[dsts[b]], ssems[rb]).wait()

                    idx_load(t + NBUF, (b + NBUF) % ISLOTS)
                    pltpu.async_copy(
                        table_hbm.at[srcs[b]], rows[rb], gsems[rb])

                    @pl.when(t >= 1)
                    def _():
                        pb = (b + 1) % NBUF
                        pltpu.make_async_copy(
                            table_hbm.at[srcs[b]], rows[pb],
                            gsems[pb]).wait()
                        pltpu.async_copy(
                            rows[pb], acc.at[dsts[(b + 3) % ISLOTS]],
                            ssems[pb], add=True)
            return carry

        lax.fori_loop(0, -(-T // ISLOTS), step4, 0)

        def last_scatter(gslot, islot):
            pltpu.make_async_copy(
                table_hbm.at[srcs[0]], rows[gslot], gsems[gslot]).wait()
            pltpu.async_copy(
                rows[gslot], acc.at[dsts[islot]], ssems[gslot], add=True)

        @pl.when(wid < 4)
        def _():
            last_scatter(0, 2)

        @pl.when(wid >= 4)
        def _():
            last_scatter(1, 1)

        for rb in range(NBUF):
            pltpu.make_async_copy(
                rows[rb], acc.at[dsts[0]], ssems[rb]).wait()

        @pl.when(wid < 4)
        def _():
            idx_wait(3)
            idx_wait(0)

        @pl.when(wid >= 4)
        def _():
            idx_wait(2)
            idx_wait(3)

        plsc.subcore_barrier()
        pltpu.sync_copy(acc.at[pl.ds(r0, ROWS_PER_TILE)],
                        out_hbm.at[cid, pl.ds(r0, ROWS_PER_TILE)])

    params = pltpu.CompilerParams(use_tc_tiling_on_sc=tiled)
    return functools.partial(
        pl.kernel, out_type=out_type, mesh=mesh, scratch_types=scratch,
        compiler_params=params, name=f"sc_spmm_{d}")(kern)


def _fill(ref, nrows, d, value):
    def outer(i, carry):
        def inner(j, carry2):
            ref[i, pl.ds(j * 16, 16)] = jnp.full((16,), value, jnp.float32)
            return carry2
        return lax.fori_loop(0, d // 16, inner, carry)
    lax.fori_loop(0, nrows, outer, 0)


def _make_sc_deg(tiled):
    d = 16
    mesh = plsc.VectorSubcoreMesh(
        core_axis_name="c", subcore_axis_name="s",
        num_cores=NC, num_subcores=NS)
    scratch = {
        "dstall": pltpu.VMEM((T, CHUNK), jnp.int32),
        "ones": pltpu.VMEM((CHUNK, d), jnp.float32),
        "zbuf": pltpu.VMEM((ZROWS, d), jnp.float32),
        "acc": pltpu.VMEM_SHARED((N_PAD, d), jnp.float32),
        "sem": pltpu.SemaphoreType.DMA,
    }
    out_type = jax.ShapeDtypeStruct((NC, N_PAD, 8), jnp.float32)

    def kern(dst2d_hbm, out_hbm, *, dstall, ones, zbuf, acc, sem):
        cid = lax.axis_index("c")
        sid = lax.axis_index("s")
        start = (cid * NS + sid) * T

        _fill(ones, CHUNK, d, 1.0)
        _fill(zbuf, ZROWS, d, 0.0)
        pltpu.sync_copy(dst2d_hbm.at[pl.ds(start, T)], dstall)
        r0 = sid * ROWS_PER_TILE
        for j in range(ROWS_PER_TILE // ZROWS):
            pltpu.sync_copy(zbuf, acc.at[pl.ds(r0 + j * ZROWS, ZROWS)])
        plsc.subcore_barrier()

        def fire(t, carry):
            pltpu.async_copy(ones, acc.at[dstall.at[t]], sem, add=True)
            return carry

        lax.fori_loop(0, T, fire, 0)

        def drain(t, carry):
            pltpu.make_async_copy(ones, acc.at[dstall.at[t]], sem).wait()
            return carry

        lax.fori_loop(0, T, drain, 0)
        plsc.subcore_barrier()
        pltpu.sync_copy(acc.at[pl.ds(r0, ROWS_PER_TILE), pl.ds(0, 8)],
                        out_hbm.at[cid, pl.ds(r0, ROWS_PER_TILE)])

    params = pltpu.CompilerParams(use_tc_tiling_on_sc=tiled)
    return functools.partial(
        pl.kernel, out_type=out_type, mesh=mesh, scratch_types=scratch,
        compiler_params=params, name="sc_deg")(kern)


_sc_deg = _make_sc_deg(tiled=False)
_sc_spmm128 = _make_sc_spmm(D_IN, tiled=True)
_sc_spmm64 = _make_sc_spmm(D_OUT, tiled=False)

_BLK = 1000
_GRID = N_NODES // _BLK


def _prescale_body(degp_ref, x_ref, dinv_ref, xs_ref):
    deg = degp_ref[0] + degp_ref[1] + 1.0
    dinv = lax.rsqrt(deg)
    dinv_ref[...] = dinv
    xs_ref[...] = dinv * x_ref[...]


_tc_prescale = pl.pallas_call(
    _prescale_body,
    grid=(_GRID,),
    in_specs=[
        pl.BlockSpec((NC, _BLK, 1), lambda i: (0, i, 0)),
        pl.BlockSpec((_BLK, D_IN), lambda i: (i, 0)),
    ],
    out_specs=[
        pl.BlockSpec((_BLK, 1), lambda i: (i, 0)),
        pl.BlockSpec((_BLK, D_IN), lambda i: (i, 0)),
    ],
    out_shape=[
        jax.ShapeDtypeStruct((N_NODES, 1), jnp.float32),
        jax.ShapeDtypeStruct((N_NODES, D_IN), jnp.float32),
    ],
)


def _mid_body(y1p_ref, xs_ref, dinv_ref, w1_ref, b1_ref, w2_ref, gs_ref):
    z = dinv_ref[...] * (y1p_ref[0] + y1p_ref[1] + xs_ref[...])
    h = jnp.dot(z, w1_ref[...], preferred_element_type=jnp.float32)
    h = jnp.maximum(h + b1_ref[...], 0.0)
    g = jnp.dot(h, w2_ref[...], preferred_element_type=jnp.float32)
    gs_ref[...] = dinv_ref[...] * g


_tc_mid = pl.pallas_call(
    _mid_body,
    grid=(_GRID,),
    in_specs=[
        pl.BlockSpec((NC, _BLK, D_IN), lambda i: (0, i, 0)),
        pl.BlockSpec((_BLK, D_IN), lambda i: (i, 0)),
        pl.BlockSpec((_BLK, 1), lambda i: (i, 0)),
        pl.BlockSpec((D_IN, D_HID), lambda i: (0, 0)),
        pl.BlockSpec((1, D_HID), lambda i: (0, 0)),
        pl.BlockSpec((D_HID, D_OUT), lambda i: (0, 0)),
    ],
    out_specs=pl.BlockSpec((_BLK, D_OUT), lambda i: (i, 0)),
    out_shape=jax.ShapeDtypeStruct((N_NODES, D_OUT), jnp.float32),
)


def _final_body(y2p_ref, gs_ref, dinv_ref, b2_ref, out_ref):
    t = dinv_ref[...] * (y2p_ref[0] + y2p_ref[1] + gs_ref[...]) + b2_ref[...]
    m = jnp.max(t, axis=1, keepdims=True)
    e = jnp.exp(t - m)
    s = jnp.sum(e, axis=1, keepdims=True)
    out_ref[...] = (t - m) - jnp.log(s)


_tc_final = pl.pallas_call(
    _final_body,
    grid=(_GRID,),
    in_specs=[
        pl.BlockSpec((NC, _BLK, D_OUT), lambda i: (0, i, 0)),
        pl.BlockSpec((_BLK, D_OUT), lambda i: (i, 0)),
        pl.BlockSpec((_BLK, 1), lambda i: (i, 0)),
        pl.BlockSpec((1, D_OUT), lambda i: (0, 0)),
    ],
    out_specs=pl.BlockSpec((_BLK, D_OUT), lambda i: (i, 0)),
    out_shape=jax.ShapeDtypeStruct((N_NODES, D_OUT), jnp.float32),
)


@jax.jit
def kernel(x, edge_index, W1, b1, W2, b2):
    npad = E_PAD - N_EDGES
    src = jnp.concatenate(
        [edge_index[0], jnp.zeros((npad,), jnp.int32)])
    dst = jnp.concatenate(
        [edge_index[1],
         DUMP_ROW0 + (jnp.arange(npad, dtype=jnp.int32) % NDUMP)])
    degp = _sc_deg(dst.reshape(CH_PAD, CHUNK))
    dinv, xs = _tc_prescale(degp[:, :N_NODES, 0:1], x)
    y1p = _sc_spmm128(xs, src, dst)
    gs = _tc_mid(y1p, xs, dinv, W1, b1.reshape(1, D_HID), W2)
    y2p = _sc_spmm64(gs, src, dst)
    return _tc_final(y2p, gs, dinv, b2.reshape(1, D_OUT))


# speedup vs baseline: 1.0369x; 1.0369x over previous
import functools

import jax
import jax.numpy as jnp
from jax import lax
from jax.experimental import pallas as pl
from jax.experimental.pallas import tpu as pltpu
from jax.experimental.pallas import tpu_sc as plsc

N_NODES = 10000
N_PAD = 10240
N_EDGES = 320000
D_IN = 128
D_HID = 256
D_OUT = 64

NC = 2
NS = 16
CHUNK = 128
NCHUNKS = N_EDGES // CHUNK
T_STEPS = -(-NCHUNKS // (NC * NS))
ROWS_PER_TILE = N_PAD // NS
ZROWS = 128


NBUF = 2
ISLOTS = 4
W = NC * NS
T = 79
CH_PAD = W * T
E_PAD = CH_PAD * CHUNK
DUMP_ROW0 = N_NODES
NDUMP = N_PAD - N_NODES


def _make_sc_spmm(d, tiled):
    mesh = plsc.VectorSubcoreMesh(
        core_axis_name="c", subcore_axis_name="s",
        num_cores=NC, num_subcores=NS)
    scratch = {
        "srcs": [pltpu.VMEM((CHUNK,), jnp.int32) for _ in range(ISLOTS)],
        "dsts": [pltpu.VMEM((CHUNK,), jnp.int32) for _ in range(ISLOTS)],
        "rows": [pltpu.VMEM((CHUNK, d), jnp.float32) for _ in range(NBUF)],
        "acc": pltpu.VMEM_SHARED((N_PAD, d), jnp.float32),
        "isems": [pltpu.SemaphoreType.DMA for _ in range(ISLOTS)],
        "gsems": [pltpu.SemaphoreType.DMA for _ in range(NBUF)],
        "ssems": [pltpu.SemaphoreType.DMA for _ in range(NBUF)],
    }
    out_type = jax.ShapeDtypeStruct((NC, N_PAD, d), jnp.float32)

    def kern(table_hbm, src_hbm, dst_hbm, out_hbm, *, srcs, dsts, rows,
             acc, isems, gsems, ssems):
        cid = lax.axis_index("c")
        sid = lax.axis_index("s")
        wid = cid * NS + sid
        start = wid * 78 + jnp.minimum(wid, 4)
        nch = jnp.where(wid < 4, 79, 78)

        _fill(rows[0], CHUNK, d, 0.0)
        r0 = sid * ROWS_PER_TILE
        for j in range(ROWS_PER_TILE // CHUNK):
            pltpu.sync_copy(rows[0], acc.at[pl.ds(r0 + j * CHUNK, CHUNK)])
        plsc.subcore_barrier()

        def idx_load(t, slot):
            base = (start + t) * CHUNK
            pltpu.async_copy(dst_hbm.at[pl.ds(base, CHUNK)], dsts[slot],
                             isems[slot])
            pltpu.async_copy(src_hbm.at[pl.ds(base, CHUNK)], srcs[slot],
                             isems[slot])

        def idx_wait(slot):
            pltpu.make_async_copy(
                src_hbm.at[pl.ds(0, CHUNK)], dsts[slot], isems[slot]).wait()
            pltpu.make_async_copy(
                src_hbm.at[pl.ds(0, CHUNK)], srcs[slot], isems[slot]).wait()

        idx_load(0, 0)
        idx_load(1, 1)

        def step4(g, carry):
            for b in range(ISLOTS):
                t = g * ISLOTS + b
                rb = b % NBUF

                @pl.when(t < nch)
                def _():
                    idx_wait(b)

                    @pl.when(t >= NBUF)
                    def _():
                        pltpu.make_async_copy(
                            rows[rb], acc.at[dsts[b]], ssems[rb]).wait()

                    idx_load(t + NBUF, (b + NBUF) % ISLOTS)
                    pltpu.async_copy(
                        table_hbm.at[srcs[b]], rows[rb], gsems[rb])

                    @pl.when(t >= 1)
                    def _():
                        pb = (b + 1) % NBUF
                        pltpu.make_async_copy(
                            table_hbm.at[srcs[b]], rows[pb],
                            gsems[pb]).wait()
                        pltpu.async_copy(
                            rows[pb], acc.at[dsts[(b + 3) % ISLOTS]],
                            ssems[pb], add=True)
            return carry

        lax.fori_loop(0, -(-T // ISLOTS), step4, 0)

        def last_scatter(gslot, islot):
            pltpu.make_async_copy(
                table_hbm.at[srcs[0]], rows[gslot], gsems[gslot]).wait()
            pltpu.async_copy(
                rows[gslot], acc.at[dsts[islot]], ssems[gslot], add=True)

        @pl.when(wid < 4)
        def _():
            last_scatter(0, 2)

        @pl.when(wid >= 4)
        def _():
            last_scatter(1, 1)

        for rb in range(NBUF):
            pltpu.make_async_copy(
                rows[rb], acc.at[dsts[0]], ssems[rb]).wait()

        @pl.when(wid < 4)
        def _():
            idx_wait(3)
            idx_wait(0)

        @pl.when(wid >= 4)
        def _():
            idx_wait(2)
            idx_wait(3)

        plsc.subcore_barrier()
        pltpu.sync_copy(acc.at[pl.ds(r0, ROWS_PER_TILE)],
                        out_hbm.at[cid, pl.ds(r0, ROWS_PER_TILE)])

    params = pltpu.CompilerParams(use_tc_tiling_on_sc=tiled)
    return functools.partial(
        pl.kernel, out_type=out_type, mesh=mesh, scratch_types=scratch,
        compiler_params=params, name=f"sc_spmm_{d}")(kern)


def _fill(ref, nrows, d, value):
    def outer(i, carry):
        def inner(j, carry2):
            ref[i, pl.ds(j * 16, 16)] = jnp.full((16,), value, jnp.float32)
            return carry2
        return lax.fori_loop(0, d // 16, inner, carry)
    lax.fori_loop(0, nrows, outer, 0)


def _make_sc_deg(tiled):
    d = 16
    mesh = plsc.VectorSubcoreMesh(
        core_axis_name="c", subcore_axis_name="s",
        num_cores=NC, num_subcores=NS)
    scratch = {
        "dstall": pltpu.VMEM((T, CHUNK), jnp.int32),
        "ones": pltpu.VMEM((CHUNK, d), jnp.float32),
        "zbuf": pltpu.VMEM((ZROWS, d), jnp.float32),
        "acc": pltpu.VMEM_SHARED((N_PAD, d), jnp.float32),
        "sem": pltpu.SemaphoreType.DMA,
    }
    out_type = jax.ShapeDtypeStruct((NC, N_PAD, 16), jnp.float32)

    def kern(dst2d_hbm, out_hbm, *, dstall, ones, zbuf, acc, sem):
        cid = lax.axis_index("c")
        sid = lax.axis_index("s")
        start = (cid * NS + sid) * T

        _fill(ones, CHUNK, d, 1.0)
        _fill(zbuf, ZROWS, d, 0.0)
        pltpu.sync_copy(dst2d_hbm.at[pl.ds(start, T)], dstall)
        r0 = sid * ROWS_PER_TILE
        for j in range(ROWS_PER_TILE // ZROWS):
            pltpu.sync_copy(zbuf, acc.at[pl.ds(r0 + j * ZROWS, ZROWS)])
        plsc.subcore_barrier()

        def fire(t, carry):
            pltpu.async_copy(ones, acc.at[dstall.at[t]], sem, add=True)
            return carry

        lax.fori_loop(0, T, fire, 0)

        def drain(t, carry):
            pltpu.make_async_copy(ones, acc.at[dstall.at[t]], sem).wait()
            return carry

        lax.fori_loop(0, T, drain, 0)
        plsc.subcore_barrier()
        pltpu.sync_copy(acc.at[pl.ds(r0, ROWS_PER_TILE)],
                        out_hbm.at[cid, pl.ds(r0, ROWS_PER_TILE)])

    params = pltpu.CompilerParams(use_tc_tiling_on_sc=tiled)
    return functools.partial(
        pl.kernel, out_type=out_type, mesh=mesh, scratch_types=scratch,
        compiler_params=params, name="sc_deg")(kern)


_sc_deg = _make_sc_deg(tiled=False)
_sc_spmm128 = _make_sc_spmm(D_IN, tiled=True)
_sc_spmm64 = _make_sc_spmm(D_OUT, tiled=False)

_BLK = 2000
_GRID = N_NODES // _BLK


def _prescale_body(degp_ref, x_ref, dinv_ref, xs_ref):
    deg = degp_ref[0] + degp_ref[1] + 1.0
    dinv = lax.rsqrt(deg)
    dinv_ref[...] = dinv
    xs_ref[...] = dinv * x_ref[...]


_tc_prescale = pl.pallas_call(
    _prescale_body,
    grid=(_GRID,),
    in_specs=[
        pl.BlockSpec((NC, _BLK, 1), lambda i: (0, i, 0)),
        pl.BlockSpec((_BLK, D_IN), lambda i: (i, 0)),
    ],
    out_specs=[
        pl.BlockSpec((_BLK, 1), lambda i: (i, 0)),
        pl.BlockSpec((_BLK, D_IN), lambda i: (i, 0)),
    ],
    out_shape=[
        jax.ShapeDtypeStruct((N_NODES, 1), jnp.float32),
        jax.ShapeDtypeStruct((N_NODES, D_IN), jnp.float32),
    ],
)


def _mid_body(y1p_ref, xs_ref, dinv_ref, w1_ref, b1_ref, w2_ref, gs_ref):
    z = dinv_ref[...] * (y1p_ref[0] + y1p_ref[1] + xs_ref[...])
    h = jnp.dot(z, w1_ref[...], preferred_element_type=jnp.float32)
    h = jnp.maximum(h + b1_ref[...], 0.0)
    g = jnp.dot(h, w2_ref[...], preferred_element_type=jnp.float32)
    gs_ref[...] = dinv_ref[...] * g


_tc_mid = pl.pallas_call(
    _mid_body,
    grid=(_GRID,),
    in_specs=[
        pl.BlockSpec((NC, _BLK, D_IN), lambda i: (0, i, 0)),
        pl.BlockSpec((_BLK, D_IN), lambda i: (i, 0)),
        pl.BlockSpec((_BLK, 1), lambda i: (i, 0)),
        pl.BlockSpec((D_IN, D_HID), lambda i: (0, 0)),
        pl.BlockSpec((1, D_HID), lambda i: (0, 0)),
        pl.BlockSpec((D_HID, D_OUT), lambda i: (0, 0)),
    ],
    out_specs=pl.BlockSpec((_BLK, D_OUT), lambda i: (i, 0)),
    out_shape=jax.ShapeDtypeStruct((N_NODES, D_OUT), jnp.float32),
)


def _final_body(y2p_ref, gs_ref, dinv_ref, b2_ref, out_ref):
    t = dinv_ref[...] * (y2p_ref[0] + y2p_ref[1] + gs_ref[...]) + b2_ref[...]
    m = jnp.max(t, axis=1, keepdims=True)
    e = jnp.exp(t - m)
    sm = jnp.sum(e, axis=1, keepdims=True)
    out_ref[...] = (t - m) - jnp.log(sm)


_tc_final = pl.pallas_call(
    _final_body,
    grid=(_GRID,),
    in_specs=[
        pl.BlockSpec((NC, _BLK, D_OUT), lambda i: (0, i, 0)),
        pl.BlockSpec((_BLK, D_OUT), lambda i: (i, 0)),
        pl.BlockSpec((_BLK, 1), lambda i: (i, 0)),
        pl.BlockSpec((1, D_OUT), lambda i: (0, 0)),
    ],
    out_specs=pl.BlockSpec((_BLK, D_OUT), lambda i: (i, 0)),
    out_shape=jax.ShapeDtypeStruct((N_NODES, D_OUT), jnp.float32),
)


@jax.jit
def kernel(x, edge_index, W1, b1, W2, b2):
    npad = E_PAD - N_EDGES
    ei = jnp.ravel(edge_index)
    src = jnp.concatenate([ei[:N_EDGES], jnp.zeros((npad,), jnp.int32)])
    dst = jnp.concatenate(
        [ei[N_EDGES:],
         DUMP_ROW0 + (jnp.arange(npad, dtype=jnp.int32) % NDUMP)])
    degp = _sc_deg(dst.reshape(CH_PAD, CHUNK))
    deg_col = jnp.ravel(degp)[::16].reshape(NC, N_PAD, 1)[:, :N_NODES]
    dinv, xs = _tc_prescale(deg_col, x)
    y1p = _sc_spmm128(xs, src, dst)
    gs = _tc_mid(y1p, xs, dinv, W1, b1.reshape(1, D_HID), W2)
    y2p = _sc_spmm64(gs, src, dst)
    return _tc_final(y2p, gs, dinv, b2.reshape(1, D_OUT))
